# baseline (speedup 1.0000x reference)
import functools

import jax
import jax.numpy as jnp
from jax import lax
from jax.experimental import pallas as pl
from jax.experimental.pallas import tpu as pltpu
from jax.experimental.pallas import tpu_sc as plsc

NUM_CORES = 2
NUM_SUBCORES = 16
NW = NUM_CORES * NUM_SUBCORES

BATCH = 4096
HIST = 200
DIM = 64
NV = 1000000
BBLK = BATCH // NW
LANES = 16


@functools.partial(
    pl.kernel,
    out_type=jax.ShapeDtypeStruct((HIST, DIM, BATCH), jnp.float32),
    mesh=plsc.VectorSubcoreMesh(core_axis_name="c", subcore_axis_name="s"),
    scratch_types=(
        [pltpu.VMEM((HIST, BBLK), jnp.int32)]
        + [pltpu.VMEM((BBLK, DIM), jnp.float32) for _ in range(2)]
        + [pltpu.VMEM((DIM, BBLK), jnp.float32) for _ in range(2)]
        + [pltpu.SemaphoreType.DMA for _ in range(4)]
    ),
    compiler_params=pltpu.CompilerParams(
        use_tc_tiling_on_sc=False, needs_layout_passes=False
    ),
)
def _gather_kernel(xt_hbm, tlin_hbm, out_hbm, idx_all, r0, r1, o0, o1,
                   sg0, sg1, sw0, sw1):
    rows = [r0, r1]
    obuf = [o0, o1]
    sg = [sg0, sg1]
    sw = [sw0, sw1]

    wid = lax.axis_index("s") * NUM_CORES + lax.axis_index("c")
    bbase = wid * BBLK
    pltpu.sync_copy(xt_hbm.at[:, pl.ds(bbase, BBLK)], idx_all)

    iota = lax.iota(jnp.int32, LANES)

    def issue_gather(h, b):
        pltpu.async_copy(tlin_hbm.at[idx_all.at[h]], rows[b], sg[b])

    def wait_gather(b):
        pltpu.make_async_copy(
            tlin_hbm.at[idx_all.at[0]], rows[b], sg[b]
        ).wait()

    def issue_write(h, b):
        pltpu.async_copy(
            obuf[b], out_hbm.at[h, :, pl.ds(bbase, BBLK)], sw[b]
        )

    def wait_write(b):
        pltpu.make_async_copy(
            obuf[b], out_hbm.at[0, :, pl.ds(0, BBLK)], sw[b]
        ).wait()

    def transpose(b):
        rb = rows[b]
        ob = obuf[b]

        for d in range(DIM):
            dvec = jnp.full((LANES,), d, jnp.int32)
            for k in range(BBLK // LANES):
                v = plsc.load_gather(rb, [iota + (LANES * k), dvec])
                ob[d, pl.ds(LANES * k, LANES)] = v

    issue_gather(0, 0)

    def hbody(j, _):
        h0 = 2 * j
        issue_gather(h0 + 1, 1)

        @pl.when(j > 0)
        def _():
            wait_write(0)

        wait_gather(0)
        transpose(0)
        issue_write(h0, 0)

        @pl.when(j < HIST // 2 - 1)
        def _():
            issue_gather(h0 + 2, 0)

        @pl.when(j > 0)
        def _():
            wait_write(1)

        wait_gather(1)
        transpose(1)
        issue_write(h0 + 1, 1)
        return ()

    lax.fori_loop(0, HIST // 2, hbody, (), unroll=False)
    wait_write(0)
    wait_write(1)


def kernel(x, table):
    out2 = _gather_kernel(x.T, table)
    return out2.transpose(2, 0, 1)


# speedup vs baseline: 1.2010x; 1.2010x over previous
import functools

import jax
import jax.numpy as jnp
from jax import lax
from jax.experimental import pallas as pl
from jax.experimental.pallas import tpu as pltpu
from jax.experimental.pallas import tpu_sc as plsc

NUM_CORES = 2
NUM_SUBCORES = 16
NW = NUM_CORES * NUM_SUBCORES

BATCH = 4096
HIST = 200
DIM = 64
NV = 1000000
BBLK = BATCH // NW
LANES = 16
HB = 2
SLOTS = HB * BBLK
NBLK = HIST // HB


@functools.partial(
    pl.kernel,
    out_type=jax.ShapeDtypeStruct((HIST, DIM, BATCH), jnp.float32),
    mesh=plsc.VectorSubcoreMesh(core_axis_name="c", subcore_axis_name="s"),
    scratch_types=(
        [pltpu.VMEM((HIST, BBLK), jnp.int32)]
        + [pltpu.VMEM((SLOTS,), jnp.int32) for _ in range(2)]
        + [pltpu.VMEM((SLOTS,), jnp.int32) for _ in range(2)]
        + [pltpu.VMEM((SLOTS, 2 * DIM), jnp.float32) for _ in range(2)]
        + [pltpu.VMEM((HB, DIM, BBLK), jnp.float32) for _ in range(2)]
        + [pltpu.SemaphoreType.DMA for _ in range(4)]
    ),
    compiler_params=pltpu.CompilerParams(needs_layout_passes=False),
)
def _gather_kernel(xt_hbm, t2_hbm, out_hbm, idx_all, ip0, ip1, hm0, hm1,
                   pr0, pr1, ob0, ob1, sg0, sg1, sw0, sw1):
    ipair = [ip0, ip1]
    hmul = [hm0, hm1]
    pair = [pr0, pr1]
    obuf = [ob0, ob1]
    sg = [sg0, sg1]
    sw = [sw0, sw1]

    wid = lax.axis_index("s") * NUM_CORES + lax.axis_index("c")
    bbase = wid * BBLK
    pltpu.sync_copy(xt_hbm.at[:, pl.ds(bbase, BBLK)], idx_all)

    iota = lax.iota(jnp.int32, LANES)

    def prep_indices(j, b):
        h0 = HB * j
        for hl in range(HB):
            for g in range(BBLK // LANES):
                v = idx_all[h0 + hl, pl.ds(LANES * g, LANES)]
                s = hl * BBLK + LANES * g
                ipair[b][pl.ds(s, LANES)] = lax.shift_right_logical(v, 1)
                hmul[b][pl.ds(s, LANES)] = (v & 1) * DIM

    def issue_gather(b):
        pltpu.async_copy(t2_hbm.at[ipair[b]], pair[b], sg[b])

    def wait_gather(b):
        pltpu.make_async_copy(t2_hbm.at[ipair[b]], pair[b], sg[b]).wait()

    def issue_write(j, b):
        pltpu.async_copy(
            obuf[b],
            out_hbm.at[pl.ds(HB * j, HB), :, pl.ds(bbase, BBLK)],
            sw[b],
        )

    def wait_write(b):
        pltpu.make_async_copy(
            obuf[b],
            out_hbm.at[pl.ds(0, HB), :, pl.ds(0, BBLK)],
            sw[b],
        ).wait()

    def transpose(b):
        pr = pair[b]
        ob = obuf[b]
        for hl in range(HB):
            for g in range(BBLK // LANES):
                s = hl * BBLK + LANES * g
                halfv = hmul[b][pl.ds(s, LANES)]
                rowv = iota + s

                def dbody(d, _):
                    v = plsc.load_gather(pr, [rowv, halfv + d])
                    ob[hl, d, pl.ds(LANES * g, LANES)] = v
                    return ()

                lax.fori_loop(0, DIM, dbody, (), unroll=False)

    prep_indices(0, 0)
    issue_gather(0)

    def jbody(j2, _):
        j = 2 * j2

        @pl.when(j + 1 < NBLK)
        def _():
            prep_indices(j + 1, 1)
            issue_gather(1)

        @pl.when(j2 > 0)
        def _():
            wait_write(0)

        wait_gather(0)
        transpose(0)
        issue_write(j, 0)

        @pl.when(j + 2 < NBLK)
        def _():
            prep_indices(j + 2, 0)
            issue_gather(0)

        @pl.when(j2 > 0)
        def _():
            wait_write(1)

        wait_gather(1)
        transpose(1)
        issue_write(j + 1, 1)
        return ()

    lax.fori_loop(0, NBLK // 2, jbody, (), unroll=False)
    wait_write(0)
    wait_write(1)


def kernel(x, table):
    t2 = table.reshape(NV // 2, 2 * DIM)
    out2 = _gather_kernel(x.T, t2)
    return out2.transpose(2, 0, 1)


# speedup vs baseline: 1.2091x; 1.0067x over previous
import functools

import jax
import jax.numpy as jnp
from jax import lax
from jax.experimental import pallas as pl
from jax.experimental.pallas import tpu as pltpu
from jax.experimental.pallas import tpu_sc as plsc

NUM_CORES = 2
NUM_SUBCORES = 16
NW = NUM_CORES * NUM_SUBCORES

BATCH = 4096
HIST = 200
DIM = 64
NV = 1000000
BBLK = BATCH // NW
LANES = 16
HB = 2
SLOTS = HB * BBLK
NBLK = HIST // HB


@functools.partial(
    pl.kernel,
    out_type=jax.ShapeDtypeStruct((HIST, DIM, BATCH), jnp.float32),
    mesh=plsc.VectorSubcoreMesh(core_axis_name="c", subcore_axis_name="s"),
    scratch_types=(
        [pltpu.VMEM((HIST, BBLK), jnp.int32)]
        + [pltpu.VMEM((SLOTS,), jnp.int32) for _ in range(2)]
        + [pltpu.VMEM((SLOTS,), jnp.int32) for _ in range(2)]
        + [pltpu.VMEM((SLOTS, 2 * DIM), jnp.float32) for _ in range(2)]
        + [pltpu.VMEM((HB, DIM, BBLK), jnp.float32) for _ in range(2)]
        + [pltpu.SemaphoreType.DMA for _ in range(4)]
    ),
    compiler_params=pltpu.CompilerParams(needs_layout_passes=False),
)
def _gather_kernel(xt_hbm, t2_hbm, out_hbm, idx_all, ip0, ip1, hm0, hm1,
                   pr0, pr1, ob0, ob1, sg0, sg1, sw0, sw1):
    ipair = [ip0, ip1]
    hmul = [hm0, hm1]
    pair = [pr0, pr1]
    obuf = [ob0, ob1]
    sg = [sg0, sg1]
    sw = [sw0, sw1]

    wid = lax.axis_index("s") * NUM_CORES + lax.axis_index("c")
    bbase = wid * BBLK
    pltpu.sync_copy(xt_hbm.at[:, pl.ds(bbase, BBLK)], idx_all)

    iota = lax.iota(jnp.int32, LANES)

    def prep_indices(j, b):
        h0 = HB * j
        for hl in range(HB):
            for g in range(BBLK // LANES):
                v = idx_all[h0 + hl, pl.ds(LANES * g, LANES)]
                s = hl * BBLK + LANES * g
                ipair[b][pl.ds(s, LANES)] = lax.shift_right_logical(v, 1)
                hmul[b][pl.ds(s, LANES)] = (v & 1) * DIM

    def issue_gather(b):
        pltpu.async_copy(t2_hbm.at[ipair[b]], pair[b], sg[b])

    def wait_gather(b):
        pltpu.make_async_copy(t2_hbm.at[ipair[b]], pair[b], sg[b]).wait()

    def issue_write(j, b):
        pltpu.async_copy(
            obuf[b],
            out_hbm.at[pl.ds(HB * j, HB), :, pl.ds(bbase, BBLK)],
            sw[b],
        )

    def wait_write(b):
        pltpu.make_async_copy(
            obuf[b],
            out_hbm.at[pl.ds(0, HB), :, pl.ds(0, BBLK)],
            sw[b],
        ).wait()

    def transpose(b):
        pr = pair[b]
        ob = obuf[b]
        groups = []
        for hl in range(HB):
            for g in range(BBLK // LANES):
                s = hl * BBLK + LANES * g
                groups.append((hl, g, hmul[b][pl.ds(s, LANES)], iota + s))

        def dbody(d, _):
            for hl, g, halfv, rowv in groups:
                v = plsc.load_gather(pr, [rowv, halfv + d])
                ob[hl, d, pl.ds(LANES * g, LANES)] = v
            return ()

        lax.fori_loop(0, DIM, dbody, (), unroll=False)

    prep_indices(0, 0)
    issue_gather(0)

    def jbody(j2, _):
        j = 2 * j2

        @pl.when(j + 1 < NBLK)
        def _():
            prep_indices(j + 1, 1)
            issue_gather(1)

        @pl.when(j2 > 0)
        def _():
            wait_write(0)

        wait_gather(0)
        transpose(0)
        issue_write(j, 0)

        @pl.when(j + 2 < NBLK)
        def _():
            prep_indices(j + 2, 0)
            issue_gather(0)

        @pl.when(j2 > 0)
        def _():
            wait_write(1)

        wait_gather(1)
        transpose(1)
        issue_write(j + 1, 1)
        return ()

    lax.fori_loop(0, NBLK // 2, jbody, (), unroll=False)
    wait_write(0)
    wait_write(1)


def kernel(x, table):
    t2 = table.reshape(NV // 2, 2 * DIM)
    out2 = _gather_kernel(x.T, t2)
    return out2.transpose(2, 0, 1)


# speedup vs baseline: 1.3664x; 1.1301x over previous
import functools

import jax
import jax.numpy as jnp
from jax import lax
from jax.experimental import pallas as pl
from jax.experimental.pallas import tpu as pltpu
from jax.experimental.pallas import tpu_sc as plsc

NUM_CORES = 2
NUM_SUBCORES = 16
NW = NUM_CORES * NUM_SUBCORES

BATCH = 4096
HIST = 200
DIM = 64
NV = 1000000
BBLK = BATCH // NW
LANES = 16
HB = 1
SLOTS = HB * BBLK
NBLK = HIST // HB
OBP = BBLK + 2


@functools.partial(
    pl.kernel,
    out_type=jax.ShapeDtypeStruct((HIST, DIM, BATCH), jnp.float32),
    mesh=plsc.VectorSubcoreMesh(core_axis_name="c", subcore_axis_name="s"),
    scratch_types=(
        [pltpu.VMEM((HIST, BBLK), jnp.int32)]
        + [pltpu.VMEM((SLOTS,), jnp.int32) for _ in range(2)]
        + [pltpu.VMEM((SLOTS, 2 * DIM), jnp.float32) for _ in range(2)]
        + [pltpu.VMEM((HB, DIM, OBP), jnp.float32) for _ in range(2)]
        + [pltpu.SemaphoreType.DMA for _ in range(4)]
    ),
    compiler_params=pltpu.CompilerParams(needs_layout_passes=False),
)
def _gather_kernel(xt_hbm, t2_hbm, out_hbm, idx_all, ip0, ip1,
                   pr0, pr1, ob0, ob1, sg0, sg1, sw0, sw1):
    ipair = [ip0, ip1]
    pair = [pr0, pr1]
    obuf = [ob0, ob1]
    sg = [sg0, sg1]
    sw = [sw0, sw1]

    wid = lax.axis_index("s") * NUM_CORES + lax.axis_index("c")
    bbase = wid * BBLK
    pltpu.sync_copy(xt_hbm.at[:, pl.ds(bbase, BBLK)], idx_all)

    iota = lax.iota(jnp.int32, LANES)

    def prep_indices(j, b):
        h0 = HB * j
        for hl in range(HB):
            for g in range(BBLK // LANES):
                v = idx_all[h0 + hl, pl.ds(LANES * g, LANES)]
                ipair[b][pl.ds(hl * BBLK + LANES * g, LANES)] = (
                    lax.shift_right_logical(v, 1)
                )

    def issue_gather(b):
        pltpu.async_copy(t2_hbm.at[ipair[b]], pair[b], sg[b])

    def wait_gather(b):
        pltpu.make_async_copy(t2_hbm.at[ipair[b]], pair[b], sg[b]).wait()

    def issue_write(j, b):
        pltpu.async_copy(
            obuf[b].at[:, :, pl.ds(0, BBLK)],
            out_hbm.at[pl.ds(HB * j, HB), :, pl.ds(bbase, BBLK)],
            sw[b],
        )

    def wait_write(b):
        pltpu.make_async_copy(
            obuf[b].at[:, :, pl.ds(0, BBLK)],
            out_hbm.at[pl.ds(0, HB), :, pl.ds(0, BBLK)],
            sw[b],
        ).wait()

    def transpose(j, b):
        pr = pair[b]
        ob = obuf[b]
        h0 = HB * j
        rowvs = [iota + (LANES * m) for m in range(DIM // LANES)]

        for hl in range(HB):

            def sgbody(sgi, _):
                s0 = sgi * LANES
                xv = idx_all[h0 + hl, pl.ds(s0, LANES)]
                for i in range(LANES):
                    blane = s0 + i
                    hs = (xv[i] & 1) * DIM
                    bvec = jnp.full((LANES,), blane, jnp.int32)
                    for m in range(DIM // LANES):
                        v = pr[hl * BBLK + blane, pl.ds(hs + LANES * m, LANES)]
                        plsc.store_scatter(ob.at[hl], [rowvs[m], bvec], v)
                return ()

            lax.fori_loop(0, BBLK // LANES, sgbody, (), unroll=False)

    prep_indices(0, 0)
    issue_gather(0)

    def jbody(j2, _):
        j = 2 * j2

        prep_indices(j + 1, 1)
        issue_gather(1)

        @pl.when(j2 > 0)
        def _():
            wait_write(0)

        wait_gather(0)
        transpose(j, 0)
        issue_write(j, 0)

        @pl.when(j + 2 < NBLK)
        def _():
            prep_indices(j + 2, 0)
            issue_gather(0)

        @pl.when(j2 > 0)
        def _():
            wait_write(1)

        wait_gather(1)
        transpose(j + 1, 1)
        issue_write(j + 1, 1)
        return ()

    lax.fori_loop(0, NBLK // 2, jbody, (), unroll=False)
    wait_write(0)
    wait_write(1)


def kernel(x, table):
    t2 = table.reshape(NV // 2, 2 * DIM)
    out2 = _gather_kernel(x.T, t2)
    return out2.transpose(2, 0, 1)


# speedup vs baseline: 1.3681x; 1.0012x over previous
import functools

import jax
import jax.numpy as jnp
from jax import lax
from jax.experimental import pallas as pl
from jax.experimental.pallas import tpu as pltpu
from jax.experimental.pallas import tpu_sc as plsc

NUM_CORES = 2
NUM_SUBCORES = 16
NW = NUM_CORES * NUM_SUBCORES

BATCH = 4096
HIST = 200
DIM = 64
NV = 1000000
BBLK = BATCH // NW
LANES = 16
HB = 1
SLOTS = HB * BBLK
NBLK = HIST // HB
OBP = BBLK + 5


@functools.partial(
    pl.kernel,
    out_type=jax.ShapeDtypeStruct((HIST, DIM, BATCH), jnp.float32),
    mesh=plsc.VectorSubcoreMesh(core_axis_name="c", subcore_axis_name="s"),
    scratch_types=(
        [pltpu.VMEM((HIST, BBLK), jnp.int32)]
        + [pltpu.VMEM((SLOTS,), jnp.int32) for _ in range(2)]
        + [pltpu.VMEM((SLOTS, 2 * DIM), jnp.float32) for _ in range(2)]
        + [pltpu.VMEM((HB, DIM, OBP), jnp.float32) for _ in range(2)]
        + [pltpu.SemaphoreType.DMA for _ in range(4)]
    ),
    compiler_params=pltpu.CompilerParams(needs_layout_passes=False),
)
def _gather_kernel(xt_hbm, t2_hbm, out_hbm, idx_all, ip0, ip1,
                   pr0, pr1, ob0, ob1, sg0, sg1, sw0, sw1):
    ipair = [ip0, ip1]
    pair = [pr0, pr1]
    obuf = [ob0, ob1]
    sg = [sg0, sg1]
    sw = [sw0, sw1]

    wid = lax.axis_index("s") * NUM_CORES + lax.axis_index("c")
    bbase = wid * BBLK
    pltpu.sync_copy(xt_hbm.at[:, pl.ds(bbase, BBLK)], idx_all)

    iota = lax.iota(jnp.int32, LANES)

    def prep_indices(j, b):
        h0 = HB * j
        for hl in range(HB):
            for g in range(BBLK // LANES):
                v = idx_all[h0 + hl, pl.ds(LANES * g, LANES)]
                ipair[b][pl.ds(hl * BBLK + LANES * g, LANES)] = (
                    lax.shift_right_logical(v, 1)
                )

    def issue_gather(b):
        pltpu.async_copy(t2_hbm.at[ipair[b]], pair[b], sg[b])

    def wait_gather(b):
        pltpu.make_async_copy(t2_hbm.at[ipair[b]], pair[b], sg[b]).wait()

    def issue_write(j, b):
        pltpu.async_copy(
            obuf[b].at[:, :, pl.ds(0, BBLK)],
            out_hbm.at[pl.ds(HB * j, HB), :, pl.ds(bbase, BBLK)],
            sw[b],
        )

    def wait_write(b):
        pltpu.make_async_copy(
            obuf[b].at[:, :, pl.ds(0, BBLK)],
            out_hbm.at[pl.ds(0, HB), :, pl.ds(0, BBLK)],
            sw[b],
        ).wait()

    def transpose(j, b):
        pr = pair[b]
        ob = obuf[b]
        h0 = HB * j
        rowvs = [iota + (LANES * m) for m in range(DIM // LANES)]

        for hl in range(HB):

            def sgbody(sgi, _):
                s0 = sgi * LANES
                xv = idx_all[h0 + hl, pl.ds(s0, LANES)]
                for i in range(LANES):
                    blane = s0 + i
                    hs = (xv[i] & 1) * DIM
                    bvec = jnp.full((LANES,), blane, jnp.int32)
                    for m in range(DIM // LANES):
                        v = pr[hl * BBLK + blane, pl.ds(hs + LANES * m, LANES)]
                        plsc.store_scatter(ob.at[hl], [rowvs[m], bvec], v)
                return ()

            lax.fori_loop(0, BBLK // LANES, sgbody, (), unroll=False)

    prep_indices(0, 0)
    issue_gather(0)

    def jbody(j2, _):
        j = 2 * j2

        prep_indices(j + 1, 1)
        issue_gather(1)

        @pl.when(j2 > 0)
        def _():
            wait_write(0)

        wait_gather(0)
        transpose(j, 0)
        issue_write(j, 0)

        @pl.when(j + 2 < NBLK)
        def _():
            prep_indices(j + 2, 0)
            issue_gather(0)

        @pl.when(j2 > 0)
        def _():
            wait_write(1)

        wait_gather(1)
        transpose(j + 1, 1)
        issue_write(j + 1, 1)
        return ()

    lax.fori_loop(0, NBLK // 2, jbody, (), unroll=False)
    wait_write(0)
    wait_write(1)


def kernel(x, table):
    t2 = table.reshape(NV // 2, 2 * DIM)
    out2 = _gather_kernel(x.T, t2)
    return out2.transpose(2, 0, 1)


# speedup vs baseline: 1.6241x; 1.1872x over previous
import functools

import jax
import jax.numpy as jnp
from jax import lax
from jax.experimental import pallas as pl
from jax.experimental.pallas import tpu as pltpu
from jax.experimental.pallas import tpu_sc as plsc

NUM_CORES = 2
NUM_SUBCORES = 16
NW = NUM_CORES * NUM_SUBCORES

BATCH = 4096
HIST = 200
DIM = 64
NV = 1000000
BBLK = BATCH // NW
LANES = 16
HB = 1
SLOTS = HB * BBLK
NBLK = HIST // HB
OBP = BBLK + 5


@functools.partial(
    pl.kernel,
    out_type=jax.ShapeDtypeStruct((HIST, DIM, BATCH), jnp.float32),
    mesh=plsc.VectorSubcoreMesh(core_axis_name="c", subcore_axis_name="s"),
    scratch_types=(
        [pltpu.VMEM((HIST, BBLK), jnp.int32)]
        + [pltpu.VMEM((SLOTS,), jnp.int32) for _ in range(2)]
        + [pltpu.VMEM((SLOTS, 2 * DIM), jnp.float32) for _ in range(2)]
        + [pltpu.VMEM((HB, DIM, OBP), jnp.float32) for _ in range(2)]
        + [pltpu.SemaphoreType.DMA for _ in range(4)]
    ),
    compiler_params=pltpu.CompilerParams(needs_layout_passes=False),
)
def _gather_kernel(xt_hbm, t2_hbm, out_hbm, idx_all, ip0, ip1,
                   pr0, pr1, ob0, ob1, sg0, sg1, sw0, sw1):
    ipair = [ip0, ip1]
    pair = [pr0, pr1]
    obuf = [ob0, ob1]
    sg = [sg0, sg1]
    sw = [sw0, sw1]

    wid = lax.axis_index("s") * NUM_CORES + lax.axis_index("c")
    bbase = wid * BBLK
    pltpu.sync_copy(xt_hbm.at[:, pl.ds(bbase, BBLK)], idx_all)

    iota = lax.iota(jnp.int32, LANES)

    def prep_indices(j, b):
        h0 = HB * j
        for hl in range(HB):
            for g in range(BBLK // LANES):
                v = idx_all[h0 + hl, pl.ds(LANES * g, LANES)]
                ipair[b][pl.ds(hl * BBLK + LANES * g, LANES)] = (
                    lax.shift_right_logical(v, 1)
                )

    def issue_gather(b):
        pltpu.async_copy(t2_hbm.at[ipair[b]], pair[b], sg[b])

    def wait_gather(b):
        pltpu.make_async_copy(t2_hbm.at[ipair[b]], pair[b], sg[b]).wait()

    def issue_write(j, b):
        pltpu.async_copy(
            obuf[b].at[:, :, pl.ds(0, BBLK)],
            out_hbm.at[pl.ds(HB * j, HB), :, pl.ds(bbase, BBLK)],
            sw[b],
        )

    def wait_write(b):
        pltpu.make_async_copy(
            obuf[b].at[:, :, pl.ds(0, BBLK)],
            out_hbm.at[pl.ds(0, HB), :, pl.ds(0, BBLK)],
            sw[b],
        ).wait()

    def transpose(j, b):
        pr = pair[b]
        ob = obuf[b]
        h0 = HB * j
        rowvs = [iota + (LANES * m) for m in range(DIM // LANES)]

        for hl in range(HB):

            @plsc.parallel_loop(0, BBLK, LANES)
            def sgbody(s0):
                xv = idx_all[h0 + hl, pl.ds(s0, LANES)]
                for i in range(LANES):
                    blane = s0 + i
                    hs = (xv[i] & 1) * DIM
                    bvec = jnp.full((LANES,), blane, jnp.int32)
                    for m in range(DIM // LANES):
                        v = pr[(hl * BBLK) + blane, pl.ds(hs + LANES * m, LANES)]
                        plsc.store_scatter(ob.at[hl], [rowvs[m], bvec], v)

    prep_indices(0, 0)
    issue_gather(0)

    def jbody(j2, _):
        j = 2 * j2

        prep_indices(j + 1, 1)
        issue_gather(1)

        @pl.when(j2 > 0)
        def _():
            wait_write(0)

        wait_gather(0)
        transpose(j, 0)
        issue_write(j, 0)

        @pl.when(j + 2 < NBLK)
        def _():
            prep_indices(j + 2, 0)
            issue_gather(0)

        @pl.when(j2 > 0)
        def _():
            wait_write(1)

        wait_gather(1)
        transpose(j + 1, 1)
        issue_write(j + 1, 1)
        return ()

    lax.fori_loop(0, NBLK // 2, jbody, (), unroll=False)
    wait_write(0)
    wait_write(1)


def kernel(x, table):
    t2 = table.reshape(NV // 2, 2 * DIM)
    out2 = _gather_kernel(x.T, t2)
    return out2.transpose(2, 0, 1)


# speedup vs baseline: 1.6363x; 1.0075x over previous
import functools

import jax
import jax.numpy as jnp
from jax import lax
from jax.experimental import pallas as pl
from jax.experimental.pallas import tpu as pltpu
from jax.experimental.pallas import tpu_sc as plsc

NUM_CORES = 2
NUM_SUBCORES = 16
NW = NUM_CORES * NUM_SUBCORES

BATCH = 4096
HIST = 200
DIM = 64
NV = 1000000
BBLK = BATCH // NW
LANES = 16
HB = 1
SLOTS = HB * BBLK
NBLK = HIST // HB
OBP = BBLK + 5


@functools.partial(
    pl.kernel,
    out_type=jax.ShapeDtypeStruct((HIST, DIM, BATCH), jnp.float32),
    mesh=plsc.VectorSubcoreMesh(core_axis_name="c", subcore_axis_name="s"),
    scratch_types=(
        [pltpu.VMEM((HIST, BBLK), jnp.int32)]
        + [pltpu.VMEM((SLOTS,), jnp.int32) for _ in range(2)]
        + [pltpu.VMEM((SLOTS, 2 * DIM), jnp.float32) for _ in range(2)]
        + [pltpu.VMEM((HB, DIM, OBP), jnp.float32) for _ in range(2)]
        + [pltpu.SemaphoreType.DMA for _ in range(4)]
    ),
    compiler_params=pltpu.CompilerParams(needs_layout_passes=False),
)
def _gather_kernel(xt_hbm, t2_hbm, out_hbm, idx_all, ip0, ip1,
                   pr0, pr1, ob0, ob1, sg0, sg1, sw0, sw1):
    ipair = [ip0, ip1]
    pair = [pr0, pr1]
    obuf = [ob0, ob1]
    sg = [sg0, sg1]
    sw = [sw0, sw1]

    wid = lax.axis_index("s") * NUM_CORES + lax.axis_index("c")
    bbase = wid * BBLK
    pltpu.sync_copy(xt_hbm.at[:, pl.ds(bbase, BBLK)], idx_all)

    iota = lax.iota(jnp.int32, LANES)

    def prep_indices(j, b):
        h0 = HB * j
        for hl in range(HB):
            for g in range(BBLK // LANES):
                v = idx_all[h0 + hl, pl.ds(LANES * g, LANES)]
                ipair[b][pl.ds(hl * BBLK + LANES * g, LANES)] = (
                    lax.shift_right_logical(v, 1)
                )

    def issue_gather(b):
        pltpu.async_copy(t2_hbm.at[ipair[b]], pair[b], sg[b])

    def wait_gather(b):
        pltpu.make_async_copy(t2_hbm.at[ipair[b]], pair[b], sg[b]).wait()

    def issue_write(j, b):
        pltpu.async_copy(
            obuf[b].at[:, :, pl.ds(0, BBLK)],
            out_hbm.at[pl.ds(HB * j, HB), :, pl.ds(bbase, BBLK)],
            sw[b],
        )

    def wait_write(b):
        pltpu.make_async_copy(
            obuf[b].at[:, :, pl.ds(0, BBLK)],
            out_hbm.at[pl.ds(0, HB), :, pl.ds(0, BBLK)],
            sw[b],
        ).wait()

    def transpose(j, b):
        pr = pair[b]
        ob = obuf[b]
        h0 = HB * j
        rowvs = [iota + (LANES * m) for m in range(DIM // LANES)]

        for hl in range(HB):

            @plsc.parallel_loop(0, BBLK, LANES, unroll=2)
            def sgbody(s0):
                xv = idx_all[h0 + hl, pl.ds(s0, LANES)]
                for i in range(LANES):
                    blane = s0 + i
                    hs = (xv[i] & 1) * DIM
                    bvec = jnp.full((LANES,), blane, jnp.int32)
                    for m in range(DIM // LANES):
                        v = pr[(hl * BBLK) + blane, pl.ds(hs + LANES * m, LANES)]
                        plsc.store_scatter(ob.at[hl], [rowvs[m], bvec], v)

    prep_indices(0, 0)
    issue_gather(0)

    def jbody(j2, _):
        j = 2 * j2

        prep_indices(j + 1, 1)
        issue_gather(1)

        @pl.when(j2 > 0)
        def _():
            wait_write(0)

        wait_gather(0)
        transpose(j, 0)
        issue_write(j, 0)

        @pl.when(j + 2 < NBLK)
        def _():
            prep_indices(j + 2, 0)
            issue_gather(0)

        @pl.when(j2 > 0)
        def _():
            wait_write(1)

        wait_gather(1)
        transpose(j + 1, 1)
        issue_write(j + 1, 1)
        return ()

    lax.fori_loop(0, NBLK // 2, jbody, (), unroll=False)
    wait_write(0)
    wait_write(1)


def kernel(x, table):
    t2 = table.reshape(NV // 2, 2 * DIM)
    out2 = _gather_kernel(x.T, t2)
    return out2.transpose(2, 0, 1)
